# baseline (speedup 1.0000x reference)
import functools

import jax
import jax.numpy as jnp
from jax.experimental import pallas as pl
from jax.experimental.pallas import tpu as pltpu
from jax.experimental.pallas import tpu_sc as plsc

_NBLK = 32
_NB = 320
_NPAD = _NBLK * _NB
_ECAP = 2048
_ECH = 128
_NCHUNK = _ECAP // _ECH
_APAD = 512
_D = 512
_DIST2 = 49.0
_EPS = 1e-5


def _gnrow(x, g, b):
    m = jnp.mean(x, axis=1, keepdims=True)
    v = jnp.mean((x - m) ** 2, axis=1, keepdims=True)
    return (x - m) / jnp.sqrt(v + _EPS) * g[None, :] + b[None, :]


def _dot(a, b):
    return jax.lax.dot_general(
        a, b, (((1,), (0,)), ((), ())),
        precision=jax.lax.Precision.DEFAULT,
        preferred_element_type=jnp.float32)


def _sc_edge_build(nx, ny, ax, ay, cs):
    mesh = plsc.VectorSubcoreMesh(core_axis_name="c", subcore_axis_name="s")

    @functools.partial(
        pl.kernel,
        mesh=mesh,
        out_type=[
            jax.ShapeDtypeStruct((_NBLK, _ECAP), jnp.int32),
            jax.ShapeDtypeStruct((_NBLK, _ECAP), jnp.int32),
            jax.ShapeDtypeStruct((_NBLK, _ECAP), jnp.float32),
            jax.ShapeDtypeStruct((_NBLK, _ECAP), jnp.float32),
            jax.ShapeDtypeStruct((_NBLK, 16), jnp.int32),
        ],
        scratch_types=[
            pltpu.VMEM((_NB,), jnp.float32),
            pltpu.VMEM((_NB,), jnp.float32),
            pltpu.VMEM((_APAD,), jnp.float32),
            pltpu.VMEM((_APAD,), jnp.float32),
            pltpu.VMEM((48,), jnp.int32),
            pltpu.VMEM((_ECAP,), jnp.int32),
            pltpu.VMEM((_ECAP,), jnp.int32),
            pltpu.VMEM((_ECAP,), jnp.float32),
            pltpu.VMEM((_ECAP,), jnp.float32),
            pltpu.VMEM((16,), jnp.int32),
            pltpu.VMEM((32,), jnp.int32),
        ],
    )
    def edge_kernel(nx_h, ny_h, ax_h, ay_h, cs_h, li_h, jj_h, dx_h, dy_h,
                    cnt_h, nxv, nyv, axv, ayv, csv, liv, jjv, dxv, dyv, offv,
                    bndv):
        c = jax.lax.axis_index("c")
        s = jax.lax.axis_index("s")
        w = s * 2 + c
        base = w * _NB
        pltpu.sync_copy(nx_h.at[pl.ds(base, _NB)], nxv)
        pltpu.sync_copy(ny_h.at[pl.ds(base, _NB)], nyv)
        pltpu.sync_copy(ax_h, axv)
        pltpu.sync_copy(ay_h, ayv)
        pltpu.sync_copy(cs_h, csv)
        offv[...] = jnp.zeros((16,), jnp.int32)
        cs0 = csv[pl.ds(0, 16)]
        cs1 = csv[pl.ds(16, 16)]
        cs2 = csv[pl.ds(32, 16)]

        lane = jax.lax.iota(jnp.int32, 16)

        def _splat(vec, j):
            return vec.at[jnp.full((16,), j, jnp.int32)].get(
                mode="promise_in_bounds")

        def node_group(g, carry):
            nxg = nxv[pl.ds(g * 16, 16)]
            nyg = nyv[pl.ds(g * 16, 16)]
            for l in range(16):
                nxn = _splat(nxg, l)
                nyn = _splat(nyg, l)
                li_s = g * 16 + l

                def _butterfly(x):
                    for dd in (1, 2, 4, 8):
                        x = x + x.at[lane ^ dd].get(
                            mode="promise_in_bounds")
                    return x

                def _one_group(h, mi, dxe, dye, t):
                    r = mi
                    for dd in (1, 2, 4, 8):
                        sh = r.at[jnp.maximum(lane - dd, 0)].get(
                            mode="promise_in_bounds")
                        r = r + jnp.where(lane >= dd, sh, 0)
                    rex = r - mi
                    perm = jnp.zeros((16,), jnp.int32)
                    for j in range(16):
                        rj = _splat(rex, j)
                        mj = _splat(mi, j)
                        perm = perm + mj * jnp.where(lane == rj, j, 0)
                    off = offv[...]
                    offs = jnp.minimum(off[0], _ECAP - 16)
                    sl = pl.ds(offs, 16)
                    liv[sl] = jnp.full((16,), li_s, jnp.int32)
                    jjv[sl] = h * 16 + perm
                    dxv[sl] = dxe.at[perm].get(mode="promise_in_bounds")
                    dyv[sl] = dye.at[perm].get(mode="promise_in_bounds")
                    offv[...] = off + t

                cvf = jnp.clip(nxn * (32.0 / 300.0), 0.0, 31.0)
                cvi = cvf.astype(jnp.int32)

                def _cslookup(v):
                    v0 = jnp.clip(v, 0, 15)
                    v1 = jnp.clip(v - 16, 0, 15)
                    v2 = jnp.clip(v - 32, 0, 15)
                    g0v = cs0.at[v0].get(mode="promise_in_bounds")
                    g1v = cs1.at[v1].get(mode="promise_in_bounds")
                    g2v = cs2.at[v2].get(mode="promise_in_bounds")
                    return jnp.where(v < 16, g0v,
                                     jnp.where(v < 32, g1v, g2v))

                lo = _cslookup(jnp.maximum(cvi - 1, 0))
                hi = _cslookup(jnp.minimum(cvi + 2, 32))
                bndv[pl.ds(0, 16)] = lo
                bndv[pl.ds(16, 16)] = hi
                lo2 = bndv[pl.ds(0, 16)]
                hi2 = bndv[pl.ds(16, 16)]
                gge0 = lo2[0] // 16
                gge1 = (hi2[0] + 15) // 16

                def actor_group(h, carry2):
                    axg = axv[pl.ds(h * 16, 16)]
                    ayg = ayv[pl.ds(h * 16, 16)]
                    dxe = nxn - axg
                    dye = nyn - ayg
                    d2 = dxe * dxe + dye * dye
                    mi = jnp.where(d2 <= _DIST2, 1, 0).astype(jnp.int32)
                    t = _butterfly(mi)

                    @pl.when(t[0] > 0)
                    def _append():
                        _one_group(h, mi, dxe, dye, t)
                    return carry2
                jax.lax.fori_loop(gge0, gge1, actor_group, 0)
            return carry
        jax.lax.fori_loop(0, _NB // 16, node_group, 0)

        offq = offv[...]

        def tailfix(i, carry):
            sl = pl.ds(i * 16, 16)
            idxs = i * 16 + lane
            good = idxs < offq
            liv[sl] = jnp.where(good, liv[sl], _NB)
            jjv[sl] = jnp.where(good, jjv[sl], _APAD - 1)
            dxv[sl] = jnp.where(good, dxv[sl], 0.0)
            dyv[sl] = jnp.where(good, dyv[sl], 0.0)
            return carry
        jax.lax.fori_loop(0, _ECAP // 16, tailfix, 0)

        pltpu.sync_copy(liv, li_h.at[w])
        pltpu.sync_copy(jjv, jj_h.at[w])
        pltpu.sync_copy(dxv, dx_h.at[w])
        pltpu.sync_copy(dyv, dy_h.at[w])
        pltpu.sync_copy(offv, cnt_h.at[w])

    return edge_kernel(nx, ny, ax, ay, cs)


def _k_pre(featp, meta8, WfT, WmT, pp, qWT, cW1bT, aWT):
    def body(feat_ref, m8_ref, wf_ref, wm_ref, pp_ref, qw_ref, cb_ref, aw_ref,
             f0_ref, qc_ref, ag_ref):
        p = pp_ref[...]
        x = _dot(feat_ref[...], wf_ref[...]) + _dot(m8_ref[...], wm_ref[...])
        f0 = jnp.maximum(_gnrow(x, p[0], p[1]), 0.0)
        f0_ref[...] = f0
        q = jnp.maximum(_gnrow(_dot(f0, qw_ref[...]), p[2], p[3]), 0.0)
        qc_ref[...] = _dot(q, cb_ref[...])
        ag_ref[...] = _dot(f0, aw_ref[...])

    big = jax.ShapeDtypeStruct((_NPAD, _D), jnp.float32)
    return pl.pallas_call(
        body,
        grid=(_NBLK,),
        in_specs=[
            pl.BlockSpec((_NB, _D), lambda i: (i, 0)),
            pl.BlockSpec((_NB, 8), lambda i: (i, 0)),
            pl.BlockSpec((_D, _D), lambda i: (0, 0)),
            pl.BlockSpec((8, _D), lambda i: (0, 0)),
            pl.BlockSpec((8, _D), lambda i: (0, 0)),
            pl.BlockSpec((_D, _D), lambda i: (0, 0)),
            pl.BlockSpec((_D, _D), lambda i: (0, 0)),
            pl.BlockSpec((_D, _D), lambda i: (0, 0)),
        ],
        out_specs=[pl.BlockSpec((_NB, _D), lambda i: (i, 0))] * 3,
        out_shape=[big, big, big],
    )(featp, meta8, WfT, WmT, pp, qWT, cW1bT, aWT)


def _k_sort(axy8, actorsp, cW1cT0, cW1cT1):
    def body(axy_ref, act_ref, c0_ref, c1_ref,
             meta_ref, csx_ref, o0_ref, o1_ref):
        axc = axy_ref[:, 0:1]
        ayc = axy_ref[:, 1:2]
        cols = jnp.clip(jnp.floor(axc * (32.0 / 300.0)), 0.0, 31.0)
        i128 = jax.lax.broadcasted_iota(jnp.int32, (_APAD, 128), 1) \
            .astype(jnp.float32)
        o = jnp.maximum(1.0 - jnp.abs(cols - i128), 0.0)
        it0 = jax.lax.broadcasted_iota(jnp.int32, (_APAD, _APAD), 0) \
            .astype(jnp.float32)
        it1 = jax.lax.broadcasted_iota(jnp.int32, (_APAD, _APAD), 1) \
            .astype(jnp.float32)
        lts = jnp.clip(it0 - it1, 0.0, 1.0)
        pe = _dot(lts, o)
        g = jnp.sum(o, axis=0, keepdims=True)
        ic0 = jax.lax.broadcasted_iota(jnp.int32, (128, 128), 0) \
            .astype(jnp.float32)
        ic1 = jax.lax.broadcasted_iota(jnp.int32, (128, 128), 1) \
            .astype(jnp.float32)
        ltm = jnp.clip(ic1 - ic0, 0.0, 1.0)
        cs = _dot(g, ltm)
        rank = jnp.sum(o * (pe + cs), axis=1, keepdims=True)
        permt = jnp.maximum(1.0 - jnp.abs(it1 - rank), 0.0)

        def bsplit(x):
            a = x.astype(jnp.bfloat16).astype(jnp.float32)
            r1 = x - a
            b = r1.astype(jnp.bfloat16).astype(jnp.float32)
            c = (r1 - b).astype(jnp.bfloat16).astype(jnp.float32)
            return a, b, c

        xa, xb, xc = bsplit(axc)
        ya, yb, yc = bsplit(ayc)
        zc = jnp.zeros((_APAD, 1), jnp.float32)
        cmat = jnp.concatenate([xa, ya, xb, yb, xc, yc, zc, zc], axis=1)

        def dott(a, b):
            return jax.lax.dot_general(
                a, b, (((0,), (0,)), ((), ())),
                preferred_element_type=jnp.float32)

        sel = dott(permt, cmat)
        axs = sel[:, 0:1] + sel[:, 2:3] + sel[:, 4:5]
        ays = sel[:, 1:2] + sel[:, 3:4] + sel[:, 5:6]
        meta_ref[...] = jnp.concatenate(
            [axs, ays, jnp.zeros((_APAD, 6), jnp.float32)], axis=1)
        csx_ref[...] = jnp.broadcast_to(cs, (8, 128))
        acs = dott(permt, act_ref[...])
        o0_ref[...] = _dot(acs, c0_ref[...])
        o1_ref[...] = _dot(acs, c1_ref[...])

    sq = jax.ShapeDtypeStruct((_APAD, _D), jnp.float32)
    m8 = jax.ShapeDtypeStruct((_APAD, 8), jnp.float32)
    c8 = jax.ShapeDtypeStruct((8, 128), jnp.float32)
    return pl.pallas_call(body, out_shape=[m8, c8, sq, sq])(
        axy8, actorsp, cW1cT0, cW1cT1)


def _k_edge_post(cntf, li3, jj3, dx3, dy3, qc, ac, epp, dW2T, cW1aT, cW2T,
                 ainit, res, lWT, pp, qWT=None, cW1bT=None, aWT=None):
    last = qWT is None

    def body(cnt_sm, li_ref, jj_ref, dx_ref, dy_ref, qc_ref, ac_ref, ep_ref,
             w2_ref, ca_ref, c2_ref, ai_ref, res_ref, lw_ref, pp_ref,
             *rest):
        if last:
            qw_ref = cb_ref = aw_ref = qcn_ref = agn_ref = None
            f_ref, acc_ref = rest
        else:
            qw_ref, cb_ref, aw_ref, f_ref, qcn_ref, agn_ref, acc_ref = rest
        b = pl.program_id(0)
        cnt = cnt_sm[b * 16]
        ep = ep_ref[...]

        def chunk_add(k):
            sl = pl.ds(k * _ECH, _ECH)
            lv = li_ref[0, 0, sl]
            jv = jj_ref[0, 0, sl]
            dxv = dx_ref[0, 0, sl]
            dyv = dy_ref[0, 0, sl]
            d1 = jnp.maximum(
                dxv[:, None] * ep[0][None, :] + dyv[:, None] * ep[1][None, :]
                + ep[2][None, :], 0.0)
            t = jnp.maximum(_gnrow(_dot(d1, w2_ref[...]), ep[3], ep[4]), 0.0)
            lvf = lv.astype(jnp.float32)
            jvf = jv.astype(jnp.float32)
            ohn = jnp.maximum(
                1.0 - jnp.abs(
                    lvf[:, None]
                    - jax.lax.broadcasted_iota(jnp.int32, (_ECH, _NB), 1)
                    .astype(jnp.float32)),
                0.0)
            oha = jnp.maximum(
                1.0 - jnp.abs(
                    jvf[:, None]
                    - jax.lax.broadcasted_iota(jnp.int32, (_ECH, _APAD), 1)
                    .astype(jnp.float32)),
                0.0)
            cpre = _dot(t, ca_ref[...]) + _dot(ohn, qc_ref[...]) \
                + _dot(oha, ac_ref[...])
            cmsg = _dot(jnp.maximum(_gnrow(cpre, ep[5], ep[6]), 0.0),
                        c2_ref[...])
            ohs = jnp.maximum(
                1.0 - jnp.abs(
                    lvf[None, :]
                    - jax.lax.broadcasted_iota(jnp.int32, (_NB, _ECH), 0)
                    .astype(jnp.float32)),
                0.0)
            acc_ref[...] += _dot(ohs, cmsg)

        acc_ref[...] = ai_ref[...]

        def chunk(k, carry):
            @pl.when(k * _ECH < cnt)
            def _():
                chunk_add(k)
            return carry
        jax.lax.fori_loop(0, _NCHUNK, chunk, 0)
        acc = acc_ref[...]
        p = pp_ref[...]
        a1 = jnp.maximum(_gnrow(acc, p[0], p[1]), 0.0)
        a2 = _gnrow(_dot(a1, lw_ref[...]), p[2], p[3])
        f = jnp.maximum(a2 + res_ref[...], 0.0)
        f_ref[...] = f
        if not last:
            q = jnp.maximum(_gnrow(_dot(f, qw_ref[...]), p[4], p[5]), 0.0)
            qcn_ref[...] = _dot(q, cb_ref[...])
            agn_ref[...] = _dot(f, aw_ref[...])

    blk = lambda b, c: (b, 0)
    cst = lambda b, c: (0, 0)
    e3 = pl.BlockSpec((1, 1, _ECAP), lambda b, c: (b, 0, 0))
    sq = pl.BlockSpec((_D, _D), cst)
    in_specs = [e3, e3, e3, e3,
                pl.BlockSpec((_NB, _D), blk),
                pl.BlockSpec((_APAD, _D), cst),
                pl.BlockSpec((8, _D), cst),
                sq, sq, sq,
                pl.BlockSpec((_NB, _D), blk),
                pl.BlockSpec((_NB, _D), blk),
                sq,
                pl.BlockSpec((8, _D), cst)]
    args = [cntf, li3, jj3, dx3, dy3, qc, ac, epp, dW2T, cW1aT, cW2T,
            ainit, res, lWT, pp]
    big = jax.ShapeDtypeStruct((_NPAD, _D), jnp.float32)
    if last:
        out_specs = pl.BlockSpec((_NB, _D), blk)
        out_shape = big
    else:
        in_specs += [sq, sq, sq]
        args += [qWT, cW1bT, aWT]
        out_specs = [pl.BlockSpec((_NB, _D), blk)] * 3
        out_shape = [big, big, big]
    grid_spec = pltpu.PrefetchScalarGridSpec(
        num_scalar_prefetch=1,
        grid=(_NBLK,),
        in_specs=in_specs,
        out_specs=out_specs,
        scratch_shapes=[pltpu.VMEM((_NB, _D), jnp.float32)],
    )
    return pl.pallas_call(body, grid_spec=grid_spec, out_shape=out_shape)(
        *args)


def kernel(feat, turn, control, intersect, node_ctrs, actors, actor_ctrs,
           node_idcs, actor_idcs, meta_W, meta_g, meta_b,
           att0_dW1, att0_db1, att0_dW2, att0_dg2, att0_db2,
           att0_qW, att0_qg, att0_qb,
           att0_cW1, att0_cg1, att0_cb1, att0_cW2,
           att0_aW, att0_ng, att0_nb, att0_lW, att0_lg, att0_lb,
           att1_dW1, att1_db1, att1_dW2, att1_dg2, att1_db2,
           att1_qW, att1_qg, att1_qb,
           att1_cW1, att1_cg1, att1_cb1, att1_cW2,
           att1_aW, att1_ng, att1_nb, att1_lW, att1_lg, att1_lb):
    n = feat.shape[0]
    a = actors.shape[0]
    z = jnp.zeros((_D,), jnp.float32)

    featp = jnp.pad(feat, ((0, _NPAD - n), (0, 0)))
    meta4 = jnp.concatenate(
        [turn, control[:, None], intersect[:, None]], axis=1)
    meta8 = jnp.pad(meta4, ((0, _NPAD - n), (0, 4)))
    nx = jnp.pad(node_ctrs[:, 0], (0, _NPAD - n), constant_values=1e9)
    ny = jnp.pad(node_ctrs[:, 1], (0, _NPAD - n), constant_values=1e9)
    ax = jnp.pad(actor_ctrs[:, 0], (0, _APAD - a), constant_values=1e9)
    ay = jnp.pad(actor_ctrs[:, 1], (0, _APAD - a), constant_values=1e9)
    actorsp = jnp.pad(actors, ((0, _APAD - a), (0, 0)))

    axy8 = jnp.stack(
        [ax, ay] + [jnp.zeros((_APAD,), jnp.float32)] * 6, axis=1)
    meta, csx, ac0, ac1 = _k_sort(axy8, actorsp,
                                  att0_cW1[:, 2 * _D:].T,
                                  att1_cW1[:, 2 * _D:].T)
    axs = meta[:, 0]
    ays = meta[:, 1]
    csi = csx[0, :48].astype(jnp.int32)

    li, jj, dxe, dye, cnt = _sc_edge_build(nx, ny, axs, ays, csi)
    li3 = li.reshape(_NBLK, 1, _ECAP)
    jj3 = jj.reshape(_NBLK, 1, _ECAP)
    dx3 = dxe.reshape(_NBLK, 1, _ECAP)
    dy3 = dye.reshape(_NBLK, 1, _ECAP)
    cntf = cnt.reshape(-1)

    WfT = meta_W[:, :_D].T
    WmT = jnp.pad(meta_W[:, _D:].T, ((0, 4), (0, 0)))

    pp1 = jnp.stack([meta_g, meta_b, att0_qg, att0_qb, z, z, z, z])
    f0, qc0, ag0 = _k_pre(featp, meta8, WfT, WmT, pp1,
                          att0_qW.T, att0_cW1[:, _D:2 * _D].T, att0_aW.T)

    epp0 = jnp.stack([att0_dW1[:, 0], att0_dW1[:, 1], att0_db1,
                      att0_dg2, att0_db2, att0_cg1, att0_cb1, z])
    pp2 = jnp.stack([att0_ng, att0_nb, att0_lg, att0_lb,
                     att1_qg, att1_qb, z, z])
    f1, qc1, ag1 = _k_edge_post(
        cntf, li3, jj3, dx3, dy3, qc0, ac0, epp0,
        att0_dW2.T, att0_cW1[:, :_D].T, att0_cW2.T, ag0, f0, att0_lW.T, pp2,
        att1_qW.T, att1_cW1[:, _D:2 * _D].T, att1_aW.T)

    epp1 = jnp.stack([att1_dW1[:, 0], att1_dW1[:, 1], att1_db1,
                      att1_dg2, att1_db2, att1_cg1, att1_cb1, z])
    pp3 = jnp.stack([att1_ng, att1_nb, att1_lg, att1_lb, z, z, z, z])
    f2 = _k_edge_post(
        cntf, li3, jj3, dx3, dy3, qc1, ac1, epp1,
        att1_dW2.T, att1_cW1[:, :_D].T, att1_cW2.T, ag1, f1, att1_lW.T, pp3)
    return f2[:n]


# speedup vs baseline: 1.3179x; 1.3179x over previous
import functools

import jax
import jax.numpy as jnp
from jax.experimental import pallas as pl
from jax.experimental.pallas import tpu as pltpu
from jax.experimental.pallas import tpu_sc as plsc

_NBLK = 32
_NB = 320
_NPAD = _NBLK * _NB
_ECAP = 2048
_ECH = 512
_NCHUNK = _ECAP // _ECH
_APAD = 512
_D = 512
_DIST2 = 49.0
_EPS = 1e-5


def _gnrow(x, g, b):
    m = jnp.mean(x, axis=1, keepdims=True)
    v = jnp.mean((x - m) ** 2, axis=1, keepdims=True)
    return (x - m) / jnp.sqrt(v + _EPS) * g[None, :] + b[None, :]


def _dot(a, b):
    return jax.lax.dot_general(
        a, b, (((1,), (0,)), ((), ())),
        precision=jax.lax.Precision.DEFAULT,
        preferred_element_type=jnp.float32)


def _sc_edge_build(nx, ny, ax, ay, cs):
    mesh = plsc.VectorSubcoreMesh(core_axis_name="c", subcore_axis_name="s")

    @functools.partial(
        pl.kernel,
        mesh=mesh,
        out_type=[
            jax.ShapeDtypeStruct((_NBLK, _ECAP), jnp.int32),
            jax.ShapeDtypeStruct((_NBLK, _ECAP), jnp.int32),
            jax.ShapeDtypeStruct((_NBLK, _ECAP), jnp.float32),
            jax.ShapeDtypeStruct((_NBLK, _ECAP), jnp.float32),
            jax.ShapeDtypeStruct((_NBLK, 16), jnp.int32),
        ],
        scratch_types=[
            pltpu.VMEM((_NB,), jnp.float32),
            pltpu.VMEM((_NB,), jnp.float32),
            pltpu.VMEM((_APAD,), jnp.float32),
            pltpu.VMEM((_APAD,), jnp.float32),
            pltpu.VMEM((48,), jnp.int32),
            pltpu.VMEM((_ECAP,), jnp.int32),
            pltpu.VMEM((_ECAP,), jnp.int32),
            pltpu.VMEM((_ECAP,), jnp.float32),
            pltpu.VMEM((_ECAP,), jnp.float32),
            pltpu.VMEM((16,), jnp.int32),
            pltpu.VMEM((32,), jnp.int32),
        ],
    )
    def edge_kernel(nx_h, ny_h, ax_h, ay_h, cs_h, li_h, jj_h, dx_h, dy_h,
                    cnt_h, nxv, nyv, axv, ayv, csv, liv, jjv, dxv, dyv, offv,
                    bndv):
        c = jax.lax.axis_index("c")
        s = jax.lax.axis_index("s")
        w = s * 2 + c
        base = w * _NB
        pltpu.sync_copy(nx_h.at[pl.ds(base, _NB)], nxv)
        pltpu.sync_copy(ny_h.at[pl.ds(base, _NB)], nyv)
        pltpu.sync_copy(ax_h, axv)
        pltpu.sync_copy(ay_h, ayv)
        pltpu.sync_copy(cs_h, csv)
        offv[...] = jnp.zeros((16,), jnp.int32)
        cs0 = csv[pl.ds(0, 16)]
        cs1 = csv[pl.ds(16, 16)]
        cs2 = csv[pl.ds(32, 16)]

        lane = jax.lax.iota(jnp.int32, 16)

        def _splat(vec, j):
            return vec.at[jnp.full((16,), j, jnp.int32)].get(
                mode="promise_in_bounds")

        def node_group(g, carry):
            nxg = nxv[pl.ds(g * 16, 16)]
            nyg = nyv[pl.ds(g * 16, 16)]
            for l in range(16):
                nxn = _splat(nxg, l)
                nyn = _splat(nyg, l)
                li_s = g * 16 + l

                def _butterfly(x):
                    for dd in (1, 2, 4, 8):
                        x = x + x.at[lane ^ dd].get(
                            mode="promise_in_bounds")
                    return x

                def _one_group(h, mi, dxe, dye, t):
                    r = mi
                    for dd in (1, 2, 4, 8):
                        sh = r.at[jnp.maximum(lane - dd, 0)].get(
                            mode="promise_in_bounds")
                        r = r + jnp.where(lane >= dd, sh, 0)
                    rex = r - mi
                    perm = jnp.zeros((16,), jnp.int32)
                    for j in range(16):
                        rj = _splat(rex, j)
                        mj = _splat(mi, j)
                        perm = perm + mj * jnp.where(lane == rj, j, 0)
                    off = offv[...]
                    offs = jnp.minimum(off[0], _ECAP - 16)
                    sl = pl.ds(offs, 16)
                    liv[sl] = jnp.full((16,), li_s, jnp.int32)
                    jjv[sl] = h * 16 + perm
                    dxv[sl] = dxe.at[perm].get(mode="promise_in_bounds")
                    dyv[sl] = dye.at[perm].get(mode="promise_in_bounds")
                    offv[...] = off + t

                cvf = jnp.clip(nxn * (32.0 / 300.0), 0.0, 31.0)
                cvi = cvf.astype(jnp.int32)

                def _cslookup(v):
                    v0 = jnp.clip(v, 0, 15)
                    v1 = jnp.clip(v - 16, 0, 15)
                    v2 = jnp.clip(v - 32, 0, 15)
                    g0v = cs0.at[v0].get(mode="promise_in_bounds")
                    g1v = cs1.at[v1].get(mode="promise_in_bounds")
                    g2v = cs2.at[v2].get(mode="promise_in_bounds")
                    return jnp.where(v < 16, g0v,
                                     jnp.where(v < 32, g1v, g2v))

                lo = _cslookup(jnp.maximum(cvi - 1, 0))
                hi = _cslookup(jnp.minimum(cvi + 2, 32))
                bndv[pl.ds(0, 16)] = lo
                bndv[pl.ds(16, 16)] = hi
                lo2 = bndv[pl.ds(0, 16)]
                hi2 = bndv[pl.ds(16, 16)]
                gge0 = lo2[0] // 16
                gge1 = (hi2[0] + 15) // 16

                def actor_group(h, carry2):
                    axg = axv[pl.ds(h * 16, 16)]
                    ayg = ayv[pl.ds(h * 16, 16)]
                    dxe = nxn - axg
                    dye = nyn - ayg
                    d2 = dxe * dxe + dye * dye
                    mi = jnp.where(d2 <= _DIST2, 1, 0).astype(jnp.int32)
                    t = _butterfly(mi)

                    @pl.when(t[0] > 0)
                    def _append():
                        _one_group(h, mi, dxe, dye, t)
                    return carry2
                jax.lax.fori_loop(gge0, gge1, actor_group, 0)
            return carry
        jax.lax.fori_loop(0, _NB // 16, node_group, 0)

        offq = offv[...]

        def tailfix(i, carry):
            sl = pl.ds(i * 16, 16)
            idxs = i * 16 + lane
            good = idxs < offq
            liv[sl] = jnp.where(good, liv[sl], _NB)
            jjv[sl] = jnp.where(good, jjv[sl], _APAD - 1)
            dxv[sl] = jnp.where(good, dxv[sl], 0.0)
            dyv[sl] = jnp.where(good, dyv[sl], 0.0)
            return carry
        jax.lax.fori_loop(0, _ECAP // 16, tailfix, 0)

        pltpu.sync_copy(liv, li_h.at[w])
        pltpu.sync_copy(jjv, jj_h.at[w])
        pltpu.sync_copy(dxv, dx_h.at[w])
        pltpu.sync_copy(dyv, dy_h.at[w])
        pltpu.sync_copy(offv, cnt_h.at[w])

    return edge_kernel(nx, ny, ax, ay, cs)


def _k_pre(featp, meta8, WfT, WmT, pp, qWT, cW1bT, aWT):
    def body(feat_ref, m8_ref, wf_ref, wm_ref, pp_ref, qw_ref, cb_ref, aw_ref,
             f0_ref, qc_ref, ag_ref):
        p = pp_ref[...]
        x = _dot(feat_ref[...], wf_ref[...]) + _dot(m8_ref[...], wm_ref[...])
        f0 = jnp.maximum(_gnrow(x, p[0], p[1]), 0.0)
        f0_ref[...] = f0
        q = jnp.maximum(_gnrow(_dot(f0, qw_ref[...]), p[2], p[3]), 0.0)
        qc_ref[...] = _dot(q, cb_ref[...])
        ag_ref[...] = _dot(f0, aw_ref[...])

    big = jax.ShapeDtypeStruct((_NPAD, _D), jnp.float32)
    return pl.pallas_call(
        body,
        grid=(_NBLK,),
        in_specs=[
            pl.BlockSpec((_NB, _D), lambda i: (i, 0)),
            pl.BlockSpec((_NB, 8), lambda i: (i, 0)),
            pl.BlockSpec((_D, _D), lambda i: (0, 0)),
            pl.BlockSpec((8, _D), lambda i: (0, 0)),
            pl.BlockSpec((8, _D), lambda i: (0, 0)),
            pl.BlockSpec((_D, _D), lambda i: (0, 0)),
            pl.BlockSpec((_D, _D), lambda i: (0, 0)),
            pl.BlockSpec((_D, _D), lambda i: (0, 0)),
        ],
        out_specs=[pl.BlockSpec((_NB, _D), lambda i: (i, 0))] * 3,
        out_shape=[big, big, big],
    )(featp, meta8, WfT, WmT, pp, qWT, cW1bT, aWT)


def _k_sort(axy8, actorsp, cW1cT0, cW1cT1):
    def body(axy_ref, act_ref, c0_ref, c1_ref,
             meta_ref, csx_ref, o0_ref, o1_ref):
        axc = axy_ref[:, 0:1]
        ayc = axy_ref[:, 1:2]
        cols = jnp.clip(jnp.floor(axc * (32.0 / 300.0)), 0.0, 31.0)
        i128 = jax.lax.broadcasted_iota(jnp.int32, (_APAD, 128), 1) \
            .astype(jnp.float32)
        o = jnp.maximum(1.0 - jnp.abs(cols - i128), 0.0)
        it0 = jax.lax.broadcasted_iota(jnp.int32, (_APAD, _APAD), 0) \
            .astype(jnp.float32)
        it1 = jax.lax.broadcasted_iota(jnp.int32, (_APAD, _APAD), 1) \
            .astype(jnp.float32)
        lts = jnp.clip(it0 - it1, 0.0, 1.0)
        pe = _dot(lts, o)
        g = jnp.sum(o, axis=0, keepdims=True)
        ic0 = jax.lax.broadcasted_iota(jnp.int32, (128, 128), 0) \
            .astype(jnp.float32)
        ic1 = jax.lax.broadcasted_iota(jnp.int32, (128, 128), 1) \
            .astype(jnp.float32)
        ltm = jnp.clip(ic1 - ic0, 0.0, 1.0)
        cs = _dot(g, ltm)
        rank = jnp.sum(o * (pe + cs), axis=1, keepdims=True)
        permt = jnp.maximum(1.0 - jnp.abs(it1 - rank), 0.0)

        def bsplit(x):
            a = x.astype(jnp.bfloat16).astype(jnp.float32)
            r1 = x - a
            b = r1.astype(jnp.bfloat16).astype(jnp.float32)
            c = (r1 - b).astype(jnp.bfloat16).astype(jnp.float32)
            return a, b, c

        xa, xb, xc = bsplit(axc)
        ya, yb, yc = bsplit(ayc)
        zc = jnp.zeros((_APAD, 1), jnp.float32)
        cmat = jnp.concatenate([xa, ya, xb, yb, xc, yc, zc, zc], axis=1)

        def dott(a, b):
            return jax.lax.dot_general(
                a, b, (((0,), (0,)), ((), ())),
                preferred_element_type=jnp.float32)

        sel = dott(permt, cmat)
        axs = sel[:, 0:1] + sel[:, 2:3] + sel[:, 4:5]
        ays = sel[:, 1:2] + sel[:, 3:4] + sel[:, 5:6]
        meta_ref[...] = jnp.concatenate(
            [axs, ays, jnp.zeros((_APAD, 6), jnp.float32)], axis=1)
        csx_ref[...] = jnp.broadcast_to(cs, (8, 128))
        acs = dott(permt, act_ref[...])
        o0_ref[...] = _dot(acs, c0_ref[...])
        o1_ref[...] = _dot(acs, c1_ref[...])

    sq = jax.ShapeDtypeStruct((_APAD, _D), jnp.float32)
    m8 = jax.ShapeDtypeStruct((_APAD, 8), jnp.float32)
    c8 = jax.ShapeDtypeStruct((8, 128), jnp.float32)
    return pl.pallas_call(body, out_shape=[m8, c8, sq, sq])(
        axy8, actorsp, cW1cT0, cW1cT1)


def _k_edge_post(cntf, li3, jj3, dx3, dy3, qc, ac, epp, dW2T, cW1aT, cW2T,
                 ainit, res, lWT, pp, qWT=None, cW1bT=None, aWT=None):
    last = qWT is None

    def body(cnt_sm, li_ref, jj_ref, dx_ref, dy_ref, qc_ref, ac_ref, ep_ref,
             w2_ref, ca_ref, c2_ref, ai_ref, res_ref, lw_ref, pp_ref,
             *rest):
        if last:
            qw_ref = cb_ref = aw_ref = qcn_ref = agn_ref = None
            f_ref, acc_ref = rest
        else:
            qw_ref, cb_ref, aw_ref, f_ref, qcn_ref, agn_ref, acc_ref = rest
        b = pl.program_id(0)
        cnt = cnt_sm[b * 16]
        ep = ep_ref[...]

        def chunk_add(k):
            sl = pl.ds(k * _ECH, _ECH)
            lv = li_ref[0, 0, sl]
            jv = jj_ref[0, 0, sl]
            dxv = dx_ref[0, 0, sl]
            dyv = dy_ref[0, 0, sl]
            d1 = jnp.maximum(
                dxv[:, None] * ep[0][None, :] + dyv[:, None] * ep[1][None, :]
                + ep[2][None, :], 0.0)
            t = jnp.maximum(_gnrow(_dot(d1, w2_ref[...]), ep[3], ep[4]), 0.0)
            lvf = lv.astype(jnp.float32)
            jvf = jv.astype(jnp.float32)
            ohn = jnp.maximum(
                1.0 - jnp.abs(
                    lvf[:, None]
                    - jax.lax.broadcasted_iota(jnp.int32, (_ECH, _NB), 1)
                    .astype(jnp.float32)),
                0.0)
            oha = jnp.maximum(
                1.0 - jnp.abs(
                    jvf[:, None]
                    - jax.lax.broadcasted_iota(jnp.int32, (_ECH, _APAD), 1)
                    .astype(jnp.float32)),
                0.0)
            cpre = _dot(t, ca_ref[...]) + _dot(ohn, qc_ref[...]) \
                + _dot(oha, ac_ref[...])
            cmsg = _dot(jnp.maximum(_gnrow(cpre, ep[5], ep[6]), 0.0),
                        c2_ref[...])
            ohs = jnp.maximum(
                1.0 - jnp.abs(
                    lvf[None, :]
                    - jax.lax.broadcasted_iota(jnp.int32, (_NB, _ECH), 0)
                    .astype(jnp.float32)),
                0.0)
            acc_ref[...] += _dot(ohs, cmsg)

        acc_ref[...] = ai_ref[...]

        def chunk(k, carry):
            @pl.when(k * _ECH < cnt)
            def _():
                chunk_add(k)
            return carry
        jax.lax.fori_loop(0, _NCHUNK, chunk, 0)
        acc = acc_ref[...]
        p = pp_ref[...]
        a1 = jnp.maximum(_gnrow(acc, p[0], p[1]), 0.0)
        a2 = _gnrow(_dot(a1, lw_ref[...]), p[2], p[3])
        f = jnp.maximum(a2 + res_ref[...], 0.0)
        f_ref[...] = f
        if not last:
            q = jnp.maximum(_gnrow(_dot(f, qw_ref[...]), p[4], p[5]), 0.0)
            qcn_ref[...] = _dot(q, cb_ref[...])
            agn_ref[...] = _dot(f, aw_ref[...])

    blk = lambda b, c: (b, 0)
    cst = lambda b, c: (0, 0)
    e3 = pl.BlockSpec((1, 1, _ECAP), lambda b, c: (b, 0, 0))
    sq = pl.BlockSpec((_D, _D), cst)
    in_specs = [e3, e3, e3, e3,
                pl.BlockSpec((_NB, _D), blk),
                pl.BlockSpec((_APAD, _D), cst),
                pl.BlockSpec((8, _D), cst),
                sq, sq, sq,
                pl.BlockSpec((_NB, _D), blk),
                pl.BlockSpec((_NB, _D), blk),
                sq,
                pl.BlockSpec((8, _D), cst)]
    args = [cntf, li3, jj3, dx3, dy3, qc, ac, epp, dW2T, cW1aT, cW2T,
            ainit, res, lWT, pp]
    big = jax.ShapeDtypeStruct((_NPAD, _D), jnp.float32)
    if last:
        out_specs = pl.BlockSpec((_NB, _D), blk)
        out_shape = big
    else:
        in_specs += [sq, sq, sq]
        args += [qWT, cW1bT, aWT]
        out_specs = [pl.BlockSpec((_NB, _D), blk)] * 3
        out_shape = [big, big, big]
    grid_spec = pltpu.PrefetchScalarGridSpec(
        num_scalar_prefetch=1,
        grid=(_NBLK,),
        in_specs=in_specs,
        out_specs=out_specs,
        scratch_shapes=[pltpu.VMEM((_NB, _D), jnp.float32)],
    )
    return pl.pallas_call(body, grid_spec=grid_spec, out_shape=out_shape)(
        *args)


def kernel(feat, turn, control, intersect, node_ctrs, actors, actor_ctrs,
           node_idcs, actor_idcs, meta_W, meta_g, meta_b,
           att0_dW1, att0_db1, att0_dW2, att0_dg2, att0_db2,
           att0_qW, att0_qg, att0_qb,
           att0_cW1, att0_cg1, att0_cb1, att0_cW2,
           att0_aW, att0_ng, att0_nb, att0_lW, att0_lg, att0_lb,
           att1_dW1, att1_db1, att1_dW2, att1_dg2, att1_db2,
           att1_qW, att1_qg, att1_qb,
           att1_cW1, att1_cg1, att1_cb1, att1_cW2,
           att1_aW, att1_ng, att1_nb, att1_lW, att1_lg, att1_lb):
    n = feat.shape[0]
    a = actors.shape[0]
    z = jnp.zeros((_D,), jnp.float32)

    featp = jnp.pad(feat, ((0, _NPAD - n), (0, 0)))
    meta4 = jnp.concatenate(
        [turn, control[:, None], intersect[:, None]], axis=1)
    meta8 = jnp.pad(meta4, ((0, _NPAD - n), (0, 4)))
    nx = jnp.pad(node_ctrs[:, 0], (0, _NPAD - n), constant_values=1e9)
    ny = jnp.pad(node_ctrs[:, 1], (0, _NPAD - n), constant_values=1e9)
    ax = jnp.pad(actor_ctrs[:, 0], (0, _APAD - a), constant_values=1e9)
    ay = jnp.pad(actor_ctrs[:, 1], (0, _APAD - a), constant_values=1e9)
    actorsp = jnp.pad(actors, ((0, _APAD - a), (0, 0)))

    axy8 = jnp.stack(
        [ax, ay] + [jnp.zeros((_APAD,), jnp.float32)] * 6, axis=1)
    meta, csx, ac0, ac1 = _k_sort(axy8, actorsp,
                                  att0_cW1[:, 2 * _D:].T,
                                  att1_cW1[:, 2 * _D:].T)
    axs = meta[:, 0]
    ays = meta[:, 1]
    csi = csx[0, :48].astype(jnp.int32)

    li, jj, dxe, dye, cnt = _sc_edge_build(nx, ny, axs, ays, csi)
    li3 = li.reshape(_NBLK, 1, _ECAP)
    jj3 = jj.reshape(_NBLK, 1, _ECAP)
    dx3 = dxe.reshape(_NBLK, 1, _ECAP)
    dy3 = dye.reshape(_NBLK, 1, _ECAP)
    cntf = cnt.reshape(-1)

    WfT = meta_W[:, :_D].T
    WmT = jnp.pad(meta_W[:, _D:].T, ((0, 4), (0, 0)))

    pp1 = jnp.stack([meta_g, meta_b, att0_qg, att0_qb, z, z, z, z])
    f0, qc0, ag0 = _k_pre(featp, meta8, WfT, WmT, pp1,
                          att0_qW.T, att0_cW1[:, _D:2 * _D].T, att0_aW.T)

    epp0 = jnp.stack([att0_dW1[:, 0], att0_dW1[:, 1], att0_db1,
                      att0_dg2, att0_db2, att0_cg1, att0_cb1, z])
    pp2 = jnp.stack([att0_ng, att0_nb, att0_lg, att0_lb,
                     att1_qg, att1_qb, z, z])
    f1, qc1, ag1 = _k_edge_post(
        cntf, li3, jj3, dx3, dy3, qc0, ac0, epp0,
        att0_dW2.T, att0_cW1[:, :_D].T, att0_cW2.T, ag0, f0, att0_lW.T, pp2,
        att1_qW.T, att1_cW1[:, _D:2 * _D].T, att1_aW.T)

    epp1 = jnp.stack([att1_dW1[:, 0], att1_dW1[:, 1], att1_db1,
                      att1_dg2, att1_db2, att1_cg1, att1_cb1, z])
    pp3 = jnp.stack([att1_ng, att1_nb, att1_lg, att1_lb, z, z, z, z])
    f2 = _k_edge_post(
        cntf, li3, jj3, dx3, dy3, qc1, ac1, epp1,
        att1_dW2.T, att1_cW1[:, :_D].T, att1_cW2.T, ag1, f1, att1_lW.T, pp3)
    return f2[:n]
